# baseline (speedup 1.0000x reference)
import jax
import jax.numpy as jnp
from jax import lax
from jax.experimental import pallas as pl
from jax.experimental.pallas import tpu as pltpu
from jax.experimental.pallas import tpu_sc as plsc

VOCAB = 1000000
D = 64
SEQ = 2048
BATCH = 16

NC = 2
NS = 16
NW = NC * NS
TOTAL = BATCH * SEQ
ROWS_PER_W = TOTAL // NW
HALF = SEQ // 2
CB = 128
NCHUNK = ROWS_PER_W // CB
SCALE = 8.0


def _body(ids_hbm, tok_hbm, posT_hbm, outT_hbm,
          ids_v, pos_v, rows_v, out_v,
          gsem0, gsem1, ssem0, ssem1):
    c_ax = lax.axis_index("c")
    s_ax = lax.axis_index("s")
    wid = s_ax * NC + c_ax
    brow = wid // 2
    half = wid % 2
    pos_base = half * HALF

    gsem = (gsem0, gsem1)
    ssem = (ssem0, ssem1)

    pltpu.sync_copy(ids_hbm.at[brow, pl.ds(pos_base, ROWS_PER_W)], ids_v)
    pltpu.sync_copy(posT_hbm.at[pl.ds(0, D), pl.ds(pos_base, ROWS_PER_W)],
                    pos_v)

    def start_fetch(c, b):
        return pltpu.async_copy(
            tok_hbm.at[ids_v.at[pl.ds(c * CB, CB)]], rows_v.at[b], gsem[b])

    def compute(c, b):
        lanes = lax.iota(jnp.int32, 16)

        @plsc.parallel_loop(0, D, step=1, unroll=2)
        def _(d):
            dvec = jnp.full((16,), d, jnp.int32)
            for k in range(CB // 16):
                rvec = lanes + (k * 16)
                g = plsc.load_gather(rows_v.at[b], [rvec, dvec])
                psl = pl.ds(c * CB + k * 16, 16)
                out_v[b, d, pl.ds(k * 16, 16)] = (g + pos_v[d, psl]) * SCALE

    fetch = [None] * NCHUNK
    store = [None] * NCHUNK
    fetch[0] = start_fetch(0, 0)
    for c in range(NCHUNK):
        b = c % 2
        if c + 1 < NCHUNK:
            if c >= 1:
                fetch[c - 1] = None
            fetch[c + 1] = start_fetch(c + 1, (c + 1) % 2)
        fetch[c].wait()
        if c >= 2:
            store[c - 2].wait()
        compute(c, b)
        store[c] = pltpu.async_copy(
            out_v.at[b],
            outT_hbm.at[brow, pl.ds(0, D), pl.ds(pos_base + c * CB, CB)],
            ssem[b])
    store[NCHUNK - 2].wait()
    store[NCHUNK - 1].wait()


@jax.jit
def _emb(ids, tok, posT):
    mesh = plsc.VectorSubcoreMesh(core_axis_name="c", subcore_axis_name="s")
    f = pl.kernel(
        _body,
        out_type=jax.ShapeDtypeStruct((BATCH, D, SEQ), jnp.float32),
        mesh=mesh,
        scratch_types=[
            pltpu.VMEM((ROWS_PER_W,), jnp.int32),
            pltpu.VMEM((D, ROWS_PER_W), jnp.float32),
            pltpu.VMEM((2, CB, D), jnp.float32),
            pltpu.VMEM((2, D, CB), jnp.float32),
            pltpu.SemaphoreType.DMA,
            pltpu.SemaphoreType.DMA,
            pltpu.SemaphoreType.DMA,
            pltpu.SemaphoreType.DMA,
        ],
        compiler_params=pltpu.CompilerParams(
            use_tc_tiling_on_sc=False, needs_layout_passes=False),
    )
    return f(ids, tok, posT)


def kernel(token_ids, token_table, pos_table):
    ids = jnp.asarray(token_ids, jnp.int32)
    outT = _emb(ids, token_table, pos_table.T)
    return outT.transpose(0, 2, 1)


# speedup vs baseline: 1.6738x; 1.6738x over previous
import jax
import jax.numpy as jnp
from jax import lax
from jax.experimental import pallas as pl
from jax.experimental.pallas import tpu as pltpu
from jax.experimental.pallas import tpu_sc as plsc

VOCAB = 1000000
D = 64
SEQ = 2048
BATCH = 16

NC = 2
NS = 16
NW = NC * NS
TOTAL = BATCH * SEQ
ROWS_PER_W = TOTAL // NW
HALF = SEQ // 2
CB = 128
NCHUNK = ROWS_PER_W // CB
SCALE = 8.0


def _body(ids_hbm, tok_hbm, posT_hbm, outT_hbm,
          ids_v, pos_v, rows_v, out_v,
          gsem0, gsem1, ssem0, ssem1):
    c_ax = lax.axis_index("c")
    s_ax = lax.axis_index("s")
    wid = s_ax * NC + c_ax
    brow = wid // 2
    half = wid % 2
    pos_base = half * HALF

    gsem = (gsem0, gsem1)
    ssem = (ssem0, ssem1)

    pltpu.sync_copy(ids_hbm.at[brow, pl.ds(pos_base, ROWS_PER_W)], ids_v)
    pltpu.sync_copy(posT_hbm.at[pl.ds(0, D), pl.ds(pos_base, ROWS_PER_W)],
                    pos_v)

    def start_fetch(c, b):
        def fire(g, _):
            tvec = ids_v[pl.ds(c * CB + g * 16, 16)]
            for l in range(16):
                tid = tvec[l]
                pltpu.async_copy(
                    tok_hbm.at[pl.ds(tid, 1)],
                    rows_v.at[b, pl.ds(g * 16 + l, 1)], gsem[b])
            return 0

        lax.fori_loop(0, CB // 16, fire, 0)

    def drain_fetch(b):
        pltpu.make_async_copy(
            tok_hbm.at[pl.ds(0, CB)], rows_v.at[b], gsem[b]).wait()

    def compute(c, b):
        lanes = lax.iota(jnp.int32, 16)

        @plsc.parallel_loop(0, D, step=1, unroll=2)
        def _(d):
            dvec = jnp.full((16,), d, jnp.int32)
            for k in range(CB // 16):
                rvec = lanes + (k * 16)
                g = plsc.load_gather(rows_v.at[b], [rvec, dvec])
                psl = pl.ds(c * CB + k * 16, 16)
                out_v[b, d, pl.ds(k * 16, 16)] = (g + pos_v[d, psl]) * SCALE

    store = [None] * NCHUNK
    start_fetch(0, 0)
    for c in range(NCHUNK):
        b = c % 2
        if c + 1 < NCHUNK:
            if c >= 1:
                store[c - 1].wait()
            start_fetch(c + 1, (c + 1) % 2)
        drain_fetch(b)
        compute(c, b)
        store[c] = pltpu.async_copy(
            out_v.at[b],
            outT_hbm.at[brow, pl.ds(0, D), pl.ds(pos_base + c * CB, CB)],
            ssem[b])
    store[NCHUNK - 2].wait()
    store[NCHUNK - 1].wait()


@jax.jit
def _emb(ids, tok, posT):
    mesh = plsc.VectorSubcoreMesh(core_axis_name="c", subcore_axis_name="s")
    f = pl.kernel(
        _body,
        out_type=jax.ShapeDtypeStruct((BATCH, D, SEQ), jnp.float32),
        mesh=mesh,
        scratch_types=[
            pltpu.VMEM((ROWS_PER_W,), jnp.int32),
            pltpu.VMEM((D, ROWS_PER_W), jnp.float32),
            pltpu.VMEM((2, CB, D), jnp.float32),
            pltpu.VMEM((2, D, CB), jnp.float32),
            pltpu.SemaphoreType.DMA,
            pltpu.SemaphoreType.DMA,
            pltpu.SemaphoreType.DMA,
            pltpu.SemaphoreType.DMA,
        ],
        compiler_params=pltpu.CompilerParams(needs_layout_passes=False),
    )
    return f(ids, tok, posT)


def kernel(token_ids, token_table, pos_table):
    ids = jnp.asarray(token_ids, jnp.int32)
    outT = _emb(ids, token_table, pos_table.T)
    return outT.transpose(0, 2, 1)


# speedup vs baseline: 1.6901x; 1.0097x over previous
import jax
import jax.numpy as jnp
from jax import lax
from jax.experimental import pallas as pl
from jax.experimental.pallas import tpu as pltpu
from jax.experimental.pallas import tpu_sc as plsc

VOCAB = 1000000
D = 64
SEQ = 2048
BATCH = 16

NC = 2
NS = 16
NW = NC * NS
TOTAL = BATCH * SEQ
ROWS_PER_W = TOTAL // NW
HALF = SEQ // 2
CB = 128
NCHUNK = ROWS_PER_W // CB
SCALE = 8.0


def _body(ids_hbm, tok_hbm, pos_hbm, out_hbm,
          ids_v, pos_v, rows_v,
          gsem0, gsem1, psem0, psem1, ssem0, ssem1):
    c_ax = lax.axis_index("c")
    s_ax = lax.axis_index("s")
    wid = s_ax * NC + c_ax
    brow = wid // 2
    half = wid % 2
    base = wid * ROWS_PER_W
    pos_base = half * HALF

    gsem = (gsem0, gsem1)
    psem = (psem0, psem1)
    ssem = (ssem0, ssem1)

    pltpu.sync_copy(ids_hbm.at[brow, pl.ds(pos_base, ROWS_PER_W)], ids_v)

    def start_fetch(c, b):
        pltpu.async_copy(
            pos_hbm.at[pl.ds(pos_base + c * CB, CB)], pos_v.at[b], psem[b])

        def fire(g, _):
            tvec = ids_v[pl.ds(c * CB + g * 16, 16)]
            for l in range(16):
                tid = tvec[l]
                pltpu.async_copy(
                    tok_hbm.at[pl.ds(tid, 1)],
                    rows_v.at[b, pl.ds(g * 16 + l, 1)], gsem[b])
            return 0

        lax.fori_loop(0, CB // 16, fire, 0)

    def drain_fetch(b):
        pltpu.make_async_copy(
            tok_hbm.at[pl.ds(0, CB)], rows_v.at[b], gsem[b]).wait()
        pltpu.make_async_copy(
            pos_hbm.at[pl.ds(0, CB)], pos_v.at[b], psem[b]).wait()

    def compute(c, b):
        @plsc.parallel_loop(0, CB, step=1, unroll=4)
        def _(j):
            for k in range(D // 16):
                sl = pl.ds(k * 16, 16)
                rows_v[b, j, sl] = (
                    rows_v[b, j, sl] + pos_v[b, j, sl]) * SCALE

    store = [None] * NCHUNK
    start_fetch(0, 0)
    for c in range(NCHUNK):
        b = c % 2
        if c + 1 < NCHUNK:
            if c >= 1:
                store[c - 1].wait()
            start_fetch(c + 1, (c + 1) % 2)
        drain_fetch(b)
        compute(c, b)
        store[c] = pltpu.async_copy(
            rows_v.at[b], out_hbm.at[pl.ds(base + c * CB, CB)], ssem[b])
    store[NCHUNK - 2].wait()
    store[NCHUNK - 1].wait()


@jax.jit
def _emb(ids, tok, pos):
    mesh = plsc.VectorSubcoreMesh(core_axis_name="c", subcore_axis_name="s")
    f = pl.kernel(
        _body,
        out_type=jax.ShapeDtypeStruct((TOTAL, D), jnp.float32),
        mesh=mesh,
        scratch_types=[
            pltpu.VMEM((ROWS_PER_W,), jnp.int32),
            pltpu.VMEM((2, CB, D), jnp.float32),
            pltpu.VMEM((2, CB, D), jnp.float32),
            pltpu.SemaphoreType.DMA,
            pltpu.SemaphoreType.DMA,
            pltpu.SemaphoreType.DMA,
            pltpu.SemaphoreType.DMA,
            pltpu.SemaphoreType.DMA,
            pltpu.SemaphoreType.DMA,
        ],
    )
    return f(ids, tok, pos)


def kernel(token_ids, token_table, pos_table):
    ids = jnp.asarray(token_ids, jnp.int32)
    out = _emb(ids, token_table, pos_table)
    return out.reshape(BATCH, SEQ, D)
